# Initial kernel scaffold; baseline (speedup 1.0000x reference)
import jax
import jax.numpy as jnp
from jax.experimental import pallas as pl


def kernel(x, token_ids, W_in, W_sel_in, W_sel_out, W_out, d_param):
    raise NotImplementedError("write your pallas kernel here")


# speedup vs baseline: 124.2223x; 124.2223x over previous
import functools

import jax
import jax.numpy as jnp
from jax.experimental import pallas as pl
from jax.experimental.pallas import tpu as pltpu
from jax.experimental.pallas import tpu_sc as plsc

B = 4
S = 2048
DIM = 1024
SD = 128
SH = 256
E = 8
BLK = 128
NBLK = 24
PADS = NBLK * BLK

_SC_NC = 2
_SC_NS = 16
_NW = _SC_NC * _SC_NS
_RPW = (B * S) // _NW
_CH = 64
_NCH = _RPW // _CH


def _plan_body(tok_ref, gpos_ref, code_ref, be_ref, used_ref):
    tok = tok_ref[...]
    xh = tok.astype(jnp.uint32)
    xh = xh ^ (xh >> 16)
    xh = xh * jnp.uint32(2246822507)
    xh = xh ^ (xh >> 13)
    xh = xh * jnp.uint32(3266489909)
    xh = xh ^ (xh >> 16)
    e = (xh & jnp.uint32(E - 1)).astype(jnp.int32)

    eids = jax.lax.broadcasted_iota(jnp.int32, (B, S, E), 2)
    oh = (e[:, :, None] == eids).astype(jnp.int32)

    c = oh
    k = 1
    while k < S:
        sh = jnp.concatenate(
            [jnp.zeros((B, k, E), jnp.int32), c[:, : S - k, :]], axis=1)
        c = c + sh
        k *= 2
    counts = c[:, S - 1, :]
    pc = ((counts + (BLK - 1)) // BLK) * BLK

    po = jnp.concatenate([jnp.zeros((B, 1), jnp.int32), pc[:, : E - 1]],
                         axis=1)
    k = 1
    while k < E:
        po = po + jnp.concatenate(
            [jnp.zeros((B, k), jnp.int32), po[:, : E - k]], axis=1)
        k *= 2
    total = po[:, E - 1 : E] + pc[:, E - 1 : E]

    sel_po = jnp.sum(oh * po[:, None, :], axis=2, dtype=jnp.int32)
    sel_cnt = jnp.sum(oh * c, axis=2, dtype=jnp.int32)
    pos = sel_po + sel_cnt - 1
    boff = jax.lax.broadcasted_iota(jnp.int32, (B, S), 0) * PADS
    gpos_ref[...] = pos + boff

    siota = jax.lax.broadcasted_iota(jnp.int32, (B, PADS), 1)
    valid = jnp.zeros((B, PADS), jnp.bool_)
    start = jnp.zeros((B, PADS), jnp.bool_)
    for j in range(E):
        poj = po[:, j : j + 1]
        cj = counts[:, j : j + 1]
        valid = valid | ((siota >= poj) & (siota < poj + cj))
        start = start | ((cj > 0) & (siota == poj))
    one = jnp.int32(1)
    two = jnp.int32(2)
    zero = jnp.int32(0)
    code_ref[...] = jnp.where(start, one, jnp.where(valid, two, zero))

    biota = jax.lax.broadcasted_iota(jnp.int32, (B, NBLK), 1) * BLK
    acc = jnp.zeros((B, NBLK), jnp.int32)
    for j in range(E):
        cond = (counts[:, j : j + 1] > 0) & (po[:, j : j + 1] <= biota)
        acc = jnp.maximum(acc, jnp.where(cond, jnp.int32(j), jnp.int32(0)))
    be_ref[...] = acc
    used_ref[...] = (biota < total).astype(jnp.int32)


def _plan(tok32):
    return pl.pallas_call(
        _plan_body,
        out_shape=(
            jax.ShapeDtypeStruct((B, S), jnp.int32),
            jax.ShapeDtypeStruct((B, PADS), jnp.int32),
            jax.ShapeDtypeStruct((B, NBLK), jnp.int32),
            jax.ShapeDtypeStruct((B, NBLK), jnp.int32),
        ),
    )(tok32)


def _shift_down(m, k, fill):
    pad = jnp.full((k, m.shape[1]), fill, m.dtype)
    return jnp.concatenate([pad, m[: m.shape[0] - k, :]], axis=0)


def _moe_body(be_s, used_s, code_ref, x_ref, win_ref, wsi_ref, wso_ref,
              wout_ref, dp_ref, out_ref, carry_ref):
    b = pl.program_id(0)
    i = pl.program_id(1)
    flat = b * NBLK + i

    @pl.when(jnp.logical_and(b == 0, i == 0))
    def _init():
        carry_ref[...] = jnp.zeros_like(carry_ref)

    @pl.when(used_s[flat] > 0)
    def _compute():
        X = x_ref[0]
        u = jnp.dot(X, win_ref[0], preferred_element_type=jnp.float32)
        sh = jnp.dot(X, wsi_ref[0], preferred_element_type=jnp.float32)
        sh = sh * jax.nn.sigmoid(sh)
        sel = jnp.dot(sh, wso_ref[0], preferred_element_type=jnp.float32)
        a_raw = sel[:, 0 * SD : 1 * SD]
        b_raw = sel[:, 1 * SD : 2 * SD]
        c_raw = sel[:, 2 * SD : 3 * SD]
        d_raw = sel[:, 3 * SD : 4 * SD]

        code = code_ref[0]
        valid = code > 0
        interior = code > 1
        a_eff = jnp.where(interior, jax.nn.sigmoid(a_raw), 0.0)
        v_eff = jnp.where(valid, jnp.tanh(b_raw) * u, 0.0)

        A, V = a_eff, v_eff
        k = 1
        while k < BLK:
            V = V + A * _shift_down(V, k, 0.0)
            A = A * _shift_down(A, k, 1.0)
            k *= 2
        carry = carry_ref[0:1, :]
        h = V + A * carry
        carry_ref[0:1, :] = h[BLK - 1 : BLK, :]

        dp = dp_ref[0]
        y = jnp.tanh(c_raw) * h + dp * jax.nn.sigmoid(d_raw) * u
        out_ref[0] = jnp.dot(y, wout_ref[0], preferred_element_type=jnp.float32)


def _moe(be_flat, used_flat, code3, x_sorted, W_in, W_sel_in, W_sel_out,
         W_out, dp3):
    def imap_x(b, i, be, used):
        return (b, i, b * 0)

    def imap_code(b, i, be, used):
        return (b * NBLK + i, b * 0, b * 0)

    def imap_w(b, i, be, used):
        return (be[b * NBLK + i], b * 0, b * 0)

    grid_spec = pltpu.PrefetchScalarGridSpec(
        num_scalar_prefetch=2,
        grid=(B, NBLK),
        in_specs=[
            pl.BlockSpec((1, BLK, 1), imap_code),
            pl.BlockSpec((1, BLK, DIM), imap_x),
            pl.BlockSpec((1, DIM, SD), imap_w),
            pl.BlockSpec((1, DIM, SH), imap_w),
            pl.BlockSpec((1, SH, 4 * SD), imap_w),
            pl.BlockSpec((1, SD, DIM), imap_w),
            pl.BlockSpec((1, 1, SD), imap_w),
        ],
        out_specs=pl.BlockSpec((1, BLK, DIM), imap_x),
        scratch_shapes=[pltpu.VMEM((8, SD), jnp.float32)],
    )
    return pl.pallas_call(
        _moe_body,
        grid_spec=grid_spec,
        out_shape=jax.ShapeDtypeStruct((B, PADS, DIM), jnp.float32),
        compiler_params=pltpu.CompilerParams(
            dimension_semantics=("arbitrary", "arbitrary")),
    )(be_flat, used_flat, code3, x_sorted, W_in, W_sel_in, W_sel_out,
      W_out, dp3)


def _sc_mesh():
    return plsc.VectorSubcoreMesh(core_axis_name="c", subcore_axis_name="s")


def _sc_scatter(x_flat, idx3):
    @functools.partial(
        pl.kernel,
        mesh=_sc_mesh(),
        out_type=jax.ShapeDtypeStruct((B * PADS, DIM), jnp.float32),
        scratch_types=[
            pltpu.VMEM((_NCH, _CH), jnp.int32),
            pltpu.VMEM((_CH, DIM), jnp.float32),
            pltpu.SemaphoreType.DMA,
        ],
    )
    def k(x_hbm, idx_hbm, out_hbm, idxv, rowv, sem):
        wid = jax.lax.axis_index("s") * _SC_NC + jax.lax.axis_index("c")
        pltpu.sync_copy(idx_hbm.at[wid], idxv)
        for j in range(_NCH):
            base = wid * _RPW + j * _CH
            pltpu.sync_copy(x_hbm.at[pl.ds(base, _CH)], rowv)
            pltpu.async_copy(rowv, out_hbm.at[idxv.at[jnp.int32(j)]],
                             sem).wait()

    return k(x_flat, idx3)


def _sc_gather(src_flat, idx3):
    @functools.partial(
        pl.kernel,
        mesh=_sc_mesh(),
        out_type=jax.ShapeDtypeStruct((B * S, DIM), jnp.float32),
        scratch_types=[
            pltpu.VMEM((_NCH, _CH), jnp.int32),
            pltpu.VMEM((_CH, DIM), jnp.float32),
            pltpu.SemaphoreType.DMA,
        ],
    )
    def k(src_hbm, idx_hbm, out_hbm, idxv, rowv, sem):
        wid = jax.lax.axis_index("s") * _SC_NC + jax.lax.axis_index("c")
        pltpu.sync_copy(idx_hbm.at[wid], idxv)
        for j in range(_NCH):
            base = wid * _RPW + j * _CH
            pltpu.async_copy(src_hbm.at[idxv.at[jnp.int32(j)]], rowv,
                             sem).wait()
            pltpu.sync_copy(rowv, out_hbm.at[pl.ds(base, _CH)])

    return k(src_flat, idx3)


def kernel(x, token_ids, W_in, W_sel_in, W_sel_out, W_out, d_param):
    tok32 = token_ids.astype(jnp.int32)
    gpos, code, be, used = _plan(tok32)

    idx3 = gpos.reshape(_NW, _NCH, _CH)
    x_sorted = _sc_scatter(x.reshape(B * S, DIM), idx3)

    out_sorted = _moe(
        be.reshape(B * NBLK),
        used.reshape(B * NBLK),
        code.reshape(B * NBLK, BLK, 1),
        x_sorted.reshape(B, PADS, DIM),
        W_in, W_sel_in, W_sel_out, W_out,
        d_param.reshape(E, 1, SD),
    )

    out = _sc_gather(out_sorted.reshape(B * PADS, DIM), idx3)
    return out.reshape(B, S, DIM)


# speedup vs baseline: 139.7876x; 1.1253x over previous
import functools

import jax
import jax.numpy as jnp
from jax.experimental import pallas as pl
from jax.experimental.pallas import tpu as pltpu
from jax.experimental.pallas import tpu_sc as plsc

B = 4
S = 2048
DIM = 1024
SD = 128
SH = 256
E = 8
BLK = 128
NBLK = 96
PADS = NBLK * BLK

_SC_NC = 2
_SC_NS = 16
_NW = _SC_NC * _SC_NS
_RPW = (B * S) // _NW
_CH = 64
_NCH = _RPW // _CH


def _plan_body(tok_ref, gpos_ref, code_ref, be_ref, used_ref):
    tok = tok_ref[...]
    xh = tok.astype(jnp.uint32)
    xh = xh ^ (xh >> 16)
    xh = xh * jnp.uint32(2246822507)
    xh = xh ^ (xh >> 13)
    xh = xh * jnp.uint32(3266489909)
    xh = xh ^ (xh >> 16)
    e = (xh & jnp.uint32(E - 1)).astype(jnp.int32)

    eids = jax.lax.broadcasted_iota(jnp.int32, (B, S, E), 2)
    oh = (e[:, :, None] == eids).astype(jnp.int32)

    c = oh
    k = 1
    while k < S:
        sh = jnp.concatenate(
            [jnp.zeros((B, k, E), jnp.int32), c[:, : S - k, :]], axis=1)
        c = c + sh
        k *= 2
    counts = c[:, S - 1, :]
    pc = ((counts + (BLK - 1)) // BLK) * BLK

    colsum = jnp.sum(pc, axis=0, keepdims=True, dtype=jnp.int32)
    ec = jnp.concatenate([jnp.zeros((1, 1), jnp.int32), colsum[:, : E - 1]],
                         axis=1)
    k = 1
    while k < E:
        ec = ec + jnp.concatenate(
            [jnp.zeros((1, k), jnp.int32), ec[:, : E - k]], axis=1)
        k *= 2
    rp = jnp.concatenate([jnp.zeros((1, E), jnp.int32), pc[: B - 1, :]],
                         axis=0)
    k = 1
    while k < B:
        rp = rp + jnp.concatenate(
            [jnp.zeros((k, E), jnp.int32), rp[: B - k, :]], axis=0)
        k *= 2
    off = ec + rp
    total = ec[:, E - 1 : E] + colsum[:, E - 1 : E]

    sel_off = jnp.sum(oh * off[:, None, :], axis=2, dtype=jnp.int32)
    sel_cnt = jnp.sum(oh * c, axis=2, dtype=jnp.int32)
    gpos_ref[...] = sel_off + sel_cnt - 1

    siota = (jax.lax.broadcasted_iota(jnp.int32, (NBLK, BLK), 0) * BLK
             + jax.lax.broadcasted_iota(jnp.int32, (NBLK, BLK), 1))
    valid = jnp.zeros((NBLK, BLK), jnp.bool_)
    start = jnp.zeros((NBLK, BLK), jnp.bool_)
    biota = jax.lax.broadcasted_iota(jnp.int32, (1, NBLK), 1) * BLK
    acc = jnp.zeros((1, NBLK), jnp.int32)
    for j in range(E):
        for b in range(B):
            offv = off[b : b + 1, j : j + 1]
            cv = counts[b : b + 1, j : j + 1]
            valid = valid | ((siota >= offv) & (siota < offv + cv))
            start = start | ((cv > 0) & (siota == offv))
            cond = (cv > 0) & (offv <= biota)
            acc = jnp.maximum(acc,
                              jnp.where(cond, jnp.int32(j), jnp.int32(0)))
    one = jnp.int32(1)
    two = jnp.int32(2)
    zero = jnp.int32(0)
    code_ref[...] = jnp.where(start, one, jnp.where(valid, two, zero))
    be_ref[...] = acc
    used_ref[...] = (biota < total).astype(jnp.int32)


def _plan(tok32):
    return pl.pallas_call(
        _plan_body,
        out_shape=(
            jax.ShapeDtypeStruct((B, S), jnp.int32),
            jax.ShapeDtypeStruct((NBLK, BLK), jnp.int32),
            jax.ShapeDtypeStruct((1, NBLK), jnp.int32),
            jax.ShapeDtypeStruct((1, NBLK), jnp.int32),
        ),
    )(tok32)


def _shift_down(m, k, fill):
    pad = jnp.full((k, m.shape[1]), fill, m.dtype)
    return jnp.concatenate([pad, m[: m.shape[0] - k, :]], axis=0)


def _moe_body(be_s, used_s, code_ref, x_ref, win_ref, wsi_ref, wso_ref,
              wout_ref, dp_ref, out_ref, carry_ref):
    i = pl.program_id(0)

    @pl.when(i == 0)
    def _init():
        carry_ref[...] = jnp.zeros_like(carry_ref)

    @pl.when(used_s[i] > 0)
    def _compute():
        X = x_ref[0]
        u = jnp.dot(X, win_ref[0], preferred_element_type=jnp.float32)
        sh = jnp.dot(X, wsi_ref[0], preferred_element_type=jnp.float32)
        sh = sh * jax.nn.sigmoid(sh)
        sel = jnp.dot(sh, wso_ref[0], preferred_element_type=jnp.float32)
        a_raw = sel[:, 0 * SD : 1 * SD]
        b_raw = sel[:, 1 * SD : 2 * SD]
        c_raw = sel[:, 2 * SD : 3 * SD]
        d_raw = sel[:, 3 * SD : 4 * SD]

        code = code_ref[0]
        valid = code > 0
        interior = code > 1
        a_eff = jnp.where(interior, jax.nn.sigmoid(a_raw), 0.0)
        v_eff = jnp.where(valid, jnp.tanh(b_raw) * u, 0.0)

        A, V = a_eff, v_eff
        k = 1
        while k < BLK:
            V = V + A * _shift_down(V, k, 0.0)
            A = A * _shift_down(A, k, 1.0)
            k *= 2
        carry = carry_ref[0:1, :]
        h = V + A * carry
        carry_ref[0:1, :] = h[BLK - 1 : BLK, :]

        dp = dp_ref[0]
        y = jnp.tanh(c_raw) * h + dp * jax.nn.sigmoid(d_raw) * u
        out_ref[0] = jnp.dot(y, wout_ref[0], preferred_element_type=jnp.float32)


def _moe(be_flat, used_flat, code3, x_sorted, W_in, W_sel_in, W_sel_out,
         W_out, dp3):
    def imap_x(i, be, used):
        return (i, i * 0, i * 0)

    def imap_w(i, be, used):
        return (be[i], i * 0, i * 0)

    grid_spec = pltpu.PrefetchScalarGridSpec(
        num_scalar_prefetch=2,
        grid=(NBLK,),
        in_specs=[
            pl.BlockSpec((1, BLK, 1), imap_x),
            pl.BlockSpec((1, BLK, DIM), imap_x),
            pl.BlockSpec((1, DIM, SD), imap_w),
            pl.BlockSpec((1, DIM, SH), imap_w),
            pl.BlockSpec((1, SH, 4 * SD), imap_w),
            pl.BlockSpec((1, SD, DIM), imap_w),
            pl.BlockSpec((1, 1, SD), imap_w),
        ],
        out_specs=pl.BlockSpec((1, BLK, DIM), imap_x),
        scratch_shapes=[pltpu.VMEM((8, SD), jnp.float32)],
    )
    return pl.pallas_call(
        _moe_body,
        grid_spec=grid_spec,
        out_shape=jax.ShapeDtypeStruct((NBLK, BLK, DIM), jnp.float32),
        compiler_params=pltpu.CompilerParams(
            dimension_semantics=("arbitrary",)),
    )(be_flat, used_flat, code3, x_sorted, W_in, W_sel_in, W_sel_out,
      W_out, dp3)


def _sc_mesh():
    return plsc.VectorSubcoreMesh(core_axis_name="c", subcore_axis_name="s")


def _sc_scatter(x_flat, idx3):
    @functools.partial(
        pl.kernel,
        mesh=_sc_mesh(),
        out_type=jax.ShapeDtypeStruct((PADS, DIM), jnp.float32),
        scratch_types=[
            pltpu.VMEM((_NCH, _CH), jnp.int32),
            pltpu.VMEM((_CH, DIM), jnp.float32),
            pltpu.SemaphoreType.DMA,
        ],
    )
    def k(x_hbm, idx_hbm, out_hbm, idxv, rowv, sem):
        wid = jax.lax.axis_index("s") * _SC_NC + jax.lax.axis_index("c")
        pltpu.sync_copy(idx_hbm.at[wid], idxv)
        for j in range(_NCH):
            base = wid * _RPW + j * _CH
            pltpu.sync_copy(x_hbm.at[pl.ds(base, _CH)], rowv)
            pltpu.async_copy(rowv, out_hbm.at[idxv.at[jnp.int32(j)]],
                             sem).wait()

    return k(x_flat, idx3)


def _sc_gather(src_flat, idx3):
    @functools.partial(
        pl.kernel,
        mesh=_sc_mesh(),
        out_type=jax.ShapeDtypeStruct((B * S, DIM), jnp.float32),
        scratch_types=[
            pltpu.VMEM((_NCH, _CH), jnp.int32),
            pltpu.VMEM((_CH, DIM), jnp.float32),
            pltpu.SemaphoreType.DMA,
        ],
    )
    def k(src_hbm, idx_hbm, out_hbm, idxv, rowv, sem):
        wid = jax.lax.axis_index("s") * _SC_NC + jax.lax.axis_index("c")
        pltpu.sync_copy(idx_hbm.at[wid], idxv)
        for j in range(_NCH):
            base = wid * _RPW + j * _CH
            pltpu.async_copy(src_hbm.at[idxv.at[jnp.int32(j)]], rowv,
                             sem).wait()
            pltpu.sync_copy(rowv, out_hbm.at[pl.ds(base, _CH)])

    return k(src_flat, idx3)


def kernel(x, token_ids, W_in, W_sel_in, W_sel_out, W_out, d_param):
    tok32 = token_ids.astype(jnp.int32)
    gpos, code, be, used = _plan(tok32)

    idx3 = gpos.reshape(_NW, _NCH, _CH)
    x_sorted = _sc_scatter(x.reshape(B * S, DIM), idx3)

    out_sorted = _moe(
        be.reshape(NBLK),
        used.reshape(NBLK),
        code.reshape(NBLK, BLK, 1),
        x_sorted.reshape(NBLK, BLK, DIM),
        W_in, W_sel_in, W_sel_out, W_out,
        d_param.reshape(E, 1, SD),
    )

    out = _sc_gather(out_sorted.reshape(PADS, DIM), idx3)
    return out.reshape(B, S, DIM)
